# baseline (speedup 1.0000x reference)
import functools

import jax
import jax.numpy as jnp
from jax import lax
from jax.experimental import pallas as pl
from jax.experimental.pallas import tpu as pltpu
from jax.experimental.pallas import tpu_sc as plsc

B, L = 1024, 100
EMB = 16
K_DIM, H1, H2, LAT = 64, 128, 64, 32

NC, NS = 2, 16
NW = NC * NS
TOT = B * L
PER_W = TOT // NW
CHUNK = 128
NCH = PER_W // CHUNK

BB = 64
GRID = B // BB

V = 1000000
CC = 8192


NP_PAIRS = 7813
VP = 128 * NP_PAIRS
NBUF = 8
TPW = (NP_PAIRS + NW - 1) // NW


def _sc_pack(table_t):
    mesh = plsc.VectorSubcoreMesh(core_axis_name="c", subcore_axis_name="s")

    @functools.partial(
        pl.kernel,
        out_type=jax.ShapeDtypeStruct((VP // 8, 128), jnp.float32),
        mesh=mesh,
        scratch_types=[
            pltpu.VMEM((NBUF, 16, 128), jnp.float32),
            pltpu.VMEM((NBUF, 16, 128), jnp.float32),
            pltpu.SemaphoreType.DMA((NBUF,)),
            pltpu.SemaphoreType.DMA((NBUF,)),
        ],
        compiler_params=pltpu.CompilerParams(
            use_tc_tiling_on_sc=False, needs_layout_passes=False),
    )
    def pack_kernel(tt_hbm, out_hbm, in_v, rows_v, sem_in, sem_out):
        wid = lax.axis_index("s") * NC + lax.axis_index("c")
        d_iota = lax.iota(jnp.int32, 16)

        def in_desc(b, c):
            return pltpu.make_async_copy(
                tt_hbm.at[:, pl.ds(128 * c, 128)], in_v.at[b], sem_in.at[b])

        def out_desc(b, c):
            return pltpu.make_async_copy(
                rows_v.at[b], out_hbm.at[pl.ds(16 * c, 16)], sem_out.at[b])

        for b in range(NBUF):
            in_desc(b, wid + NW * b).start()

        def step(t, carry):
            b = lax.rem(t, NBUF)
            c = wid + NW * t

            @pl.when(c < NP_PAIRS)
            def _():
                in_desc(b, c).wait()

                @pl.when(t >= NBUF)
                def _():
                    out_desc(b, c - NW * NBUF).wait()

                for r in range(16):
                    for jj in range(8):
                        vals = plsc.load_gather(
                            in_v.at[b],
                            [d_iota, jnp.full((16,), 8 * r + jj, jnp.int32)])
                        rows_v[b, r, pl.ds(16 * jj, 16)] = vals
                out_desc(b, c).start()

                @pl.when(c + NW * NBUF < NP_PAIRS)
                def _():
                    in_desc(b, c + NW * NBUF).start()
            return carry

        lax.fori_loop(0, TPW, step, 0, unroll=False)

        t_max = (NP_PAIRS - 1 - wid) // NW
        for b in range(NBUF):
            t_last = t_max - lax.rem(t_max - b + NBUF, NBUF)
            out_desc(b, wid + NW * t_last).wait()

    return pack_kernel(table_t)


def _sc_gather(x32, table):
    mesh = plsc.VectorSubcoreMesh(core_axis_name="c", subcore_axis_name="s")

    @functools.partial(
        pl.kernel,
        out_type=jax.ShapeDtypeStruct((NW, PER_W, EMB), jnp.float32),
        mesh=mesh,
        scratch_types=[
            pltpu.VMEM((NCH, CHUNK), jnp.int32),
            pltpu.VMEM((PER_W, EMB), jnp.float32),
            pltpu.SemaphoreType.DMA,
        ],
        compiler_params=pltpu.CompilerParams(use_tc_tiling_on_sc=False),
    )
    def gather_kernel(x_hbm, table_hbm, out_hbm, idx_v, rows_v, sem):
        wid = lax.axis_index("s") * NC + lax.axis_index("c")
        pltpu.sync_copy(x_hbm.at[wid], idx_v)

        def fire(j, _):
            pltpu.make_async_copy(
                table_hbm.at[idx_v.at[j]],
                rows_v.at[pl.ds(j * CHUNK, CHUNK)],
                sem,
            ).start()
            return _

        lax.fori_loop(0, NCH, fire, 0, unroll=False)

        def drain(j, _):
            pltpu.make_async_copy(
                table_hbm.at[idx_v.at[j]],
                rows_v.at[pl.ds(j * CHUNK, CHUNK)],
                sem,
            ).wait()
            return _

        lax.fori_loop(0, NCH, drain, 0, unroll=False)
        pltpu.sync_copy(rows_v, out_hbm.at[wid])

    return gather_kernel(x32, table)


RPB = BB * L // 8


def _tc_fused(e8, mjs3, wbig, bbig, w1, b1, w2, b2, wmu, bmu, wlv, blv, eps):
    def body(e_ref, m_ref, wb_ref, bb_ref, w1_ref, b1_ref, w2_ref, b2_ref,
             wmu_ref, bmu_ref, wlv_ref, blv_ref, eps_ref,
             z_ref, mu_ref, lv_ref):
        hp = jax.lax.Precision.HIGHEST
        p8 = jnp.dot(e_ref[...], wb_ref[...]) + bb_ref[...]
        p8 = jnp.maximum(p8, 0.0)
        bb_lo = jax.lax.broadcasted_iota(jnp.int32, (BB, RPB), 0) * L
        el8 = jax.lax.broadcasted_iota(jnp.int32, (BB, RPB), 1) * 8
        pnc = jnp.zeros((BB, K_DIM), jnp.float32)
        for j in range(8):
            el = el8 + j
            sel = ((el >= bb_lo) & (el < bb_lo + L)).astype(jnp.float32)
            sel = sel * m_ref[0, j, :][None, :]
            pnc = pnc + jnp.dot(sel, p8[:, 64 * j:64 * (j + 1)])
        h = jnp.maximum(jnp.dot(pnc, w1_ref[...], precision=hp) + b1_ref[...], 0.0)
        h = jnp.maximum(jnp.dot(h, w2_ref[...], precision=hp) + b2_ref[...], 0.0)
        mu = jnp.dot(h, wmu_ref[...], precision=hp) + bmu_ref[...]
        lv = jnp.dot(h, wlv_ref[...], precision=hp) + blv_ref[...]
        z = mu + eps_ref[...] * jnp.exp(0.5 * lv)
        z_ref[...] = z
        mu_ref[...] = mu
        lv_ref[...] = lv

    rep = lambda shape: pl.BlockSpec(shape, lambda i: (0,) * len(shape))
    out_sds = jax.ShapeDtypeStruct((B, LAT), jnp.float32)
    return pl.pallas_call(
        body,
        grid=(GRID,),
        in_specs=[
            pl.BlockSpec((RPB, 128), lambda i: (i, 0)),
            pl.BlockSpec((1, 8, RPB), lambda i: (i, 0, 0)),
            rep((128, 8 * K_DIM)), rep((1, 8 * K_DIM)),
            rep((K_DIM, H1)), rep((1, H1)),
            rep((H1, H2)), rep((1, H2)),
            rep((H2, LAT)), rep((1, LAT)),
            rep((H2, LAT)), rep((1, LAT)),
            pl.BlockSpec((BB, LAT), lambda i: (i, 0)),
        ],
        out_specs=[
            pl.BlockSpec((BB, LAT), lambda i: (i, 0)),
            pl.BlockSpec((BB, LAT), lambda i: (i, 0)),
            pl.BlockSpec((BB, LAT), lambda i: (i, 0)),
        ],
        out_shape=[out_sds, out_sds, out_sds],
    )(e8, mjs3, wbig, bbig, w1, b1, w2, b2, wmu, bmu, wlv, blv, eps)


def kernel(x, mask, table, W_pnnn, b_pnnn, W1, b1, W2, b2, Wmu, bmu, Wlv, blv, eps):
    x32 = x.reshape(NW, NCH, CHUNK)
    table_lin = _sc_pack(table.T).reshape(VP, EMB)
    e8 = _sc_gather(x32, table_lin).reshape(TOT // 8, 128)
    mjs3 = mask.astype(jnp.float32).reshape(GRID, RPB, 8).transpose(0, 2, 1)
    wbig = jnp.kron(jnp.eye(8, dtype=jnp.float32), W_pnnn.T)
    bbig = jnp.tile(b_pnnn, 8).reshape(1, 8 * K_DIM)
    z, mu, lv = _tc_fused(
        e8, mjs3, wbig, bbig,
        W1.T, b1.reshape(1, H1),
        W2.T, b2.reshape(1, H2),
        Wmu.T, bmu.reshape(1, LAT),
        Wlv.T, blv.reshape(1, LAT),
        eps,
    )
    return (z, mu, lv)


# speedup vs baseline: 3.7741x; 3.7741x over previous
import functools

import jax
import jax.numpy as jnp
from jax import lax
from jax.experimental import pallas as pl
from jax.experimental.pallas import tpu as pltpu
from jax.experimental.pallas import tpu_sc as plsc

B, L = 1024, 100
EMB = 16
K_DIM, H1, H2, LAT = 64, 128, 64, 32

NC, NS = 2, 16
NW = NC * NS
TOT = B * L
PER_W = TOT // NW
CHUNK = 128
NCH = PER_W // CHUNK

BB = 64
GRID = B // BB

V = 1000000
CC = 8192


NP_PAIRS = 7813
VP = 128 * NP_PAIRS
NBUF = 8
TPW = (NP_PAIRS + NW - 1) // NW


def _sc_pack(table_t):
    mesh = plsc.VectorSubcoreMesh(core_axis_name="c", subcore_axis_name="s")

    @functools.partial(
        pl.kernel,
        out_type=jax.ShapeDtypeStruct((VP // 8, 128), jnp.float32),
        mesh=mesh,
        scratch_types=[
            pltpu.VMEM((NBUF, 16, 128), jnp.float32),
            pltpu.VMEM((NBUF, 16, 128), jnp.float32),
            pltpu.SemaphoreType.DMA((NBUF,)),
            pltpu.SemaphoreType.DMA((NBUF,)),
        ],
        compiler_params=pltpu.CompilerParams(
            use_tc_tiling_on_sc=True, needs_layout_passes=False),
    )
    def pack_kernel(tt_hbm, out_hbm, in_v, rows_v, sem_in, sem_out):
        wid = lax.axis_index("s") * NC + lax.axis_index("c")
        d_iota = lax.iota(jnp.int32, 16)

        def in_desc(b, c):
            return pltpu.make_async_copy(
                tt_hbm.at[:, pl.ds(128 * c, 128)], in_v.at[b], sem_in.at[b])

        def out_desc(b, c):
            return pltpu.make_async_copy(
                rows_v.at[b], out_hbm.at[pl.ds(16 * c, 16)], sem_out.at[b])

        for b in range(NBUF):
            in_desc(b, wid + NW * b).start()

        def step(t, carry):
            b = lax.rem(t, NBUF)
            c = wid + NW * t

            @pl.when(c < NP_PAIRS)
            def _():
                in_desc(b, c).wait()

                @pl.when(t >= NBUF)
                def _():
                    out_desc(b, c - NW * NBUF).wait()

                for r in range(16):
                    for jj in range(8):
                        vals = plsc.load_gather(
                            in_v.at[b],
                            [d_iota, jnp.full((16,), 8 * r + jj, jnp.int32)])
                        rows_v[b, r, pl.ds(16 * jj, 16)] = vals
                out_desc(b, c).start()

                @pl.when(c + NW * NBUF < NP_PAIRS)
                def _():
                    in_desc(b, c + NW * NBUF).start()
            return carry

        lax.fori_loop(0, TPW, step, 0, unroll=False)

        t_max = (NP_PAIRS - 1 - wid) // NW
        for b in range(NBUF):
            t_last = t_max - lax.rem(t_max - b + NBUF, NBUF)
            out_desc(b, wid + NW * t_last).wait()

    return pack_kernel(table_t)


def _sc_gather(x32, table):
    mesh = plsc.VectorSubcoreMesh(core_axis_name="c", subcore_axis_name="s")

    @functools.partial(
        pl.kernel,
        out_type=jax.ShapeDtypeStruct((NW, PER_W, EMB), jnp.float32),
        mesh=mesh,
        scratch_types=[
            pltpu.VMEM((NCH, CHUNK), jnp.int32),
            pltpu.VMEM((PER_W, EMB), jnp.float32),
            pltpu.SemaphoreType.DMA,
        ],
        compiler_params=pltpu.CompilerParams(use_tc_tiling_on_sc=False),
    )
    def gather_kernel(x_hbm, table_hbm, out_hbm, idx_v, rows_v, sem):
        wid = lax.axis_index("s") * NC + lax.axis_index("c")
        pltpu.sync_copy(x_hbm.at[wid], idx_v)

        def fire(j, _):
            pltpu.make_async_copy(
                table_hbm.at[idx_v.at[j]],
                rows_v.at[pl.ds(j * CHUNK, CHUNK)],
                sem,
            ).start()
            return _

        lax.fori_loop(0, NCH, fire, 0, unroll=False)

        def drain(j, _):
            pltpu.make_async_copy(
                table_hbm.at[idx_v.at[j]],
                rows_v.at[pl.ds(j * CHUNK, CHUNK)],
                sem,
            ).wait()
            return _

        lax.fori_loop(0, NCH, drain, 0, unroll=False)
        pltpu.sync_copy(rows_v, out_hbm.at[wid])

    return gather_kernel(x32, table)


RPB = BB * L // 8


def _tc_fused(e8, mjs3, wbig, bbig, w1, b1, w2, b2, wmu, bmu, wlv, blv, eps):
    def body(e_ref, m_ref, wb_ref, bb_ref, w1_ref, b1_ref, w2_ref, b2_ref,
             wmu_ref, bmu_ref, wlv_ref, blv_ref, eps_ref,
             z_ref, mu_ref, lv_ref):
        hp = jax.lax.Precision.HIGHEST
        p8 = jnp.dot(e_ref[...], wb_ref[...]) + bb_ref[...]
        p8 = jnp.maximum(p8, 0.0)
        bb_lo = jax.lax.broadcasted_iota(jnp.int32, (BB, RPB), 0) * L
        el8 = jax.lax.broadcasted_iota(jnp.int32, (BB, RPB), 1) * 8
        pnc = jnp.zeros((BB, K_DIM), jnp.float32)
        for j in range(8):
            el = el8 + j
            sel = ((el >= bb_lo) & (el < bb_lo + L)).astype(jnp.float32)
            sel = sel * m_ref[0, j, :][None, :]
            pnc = pnc + jnp.dot(sel, p8[:, 64 * j:64 * (j + 1)])
        h = jnp.maximum(jnp.dot(pnc, w1_ref[...], precision=hp) + b1_ref[...], 0.0)
        h = jnp.maximum(jnp.dot(h, w2_ref[...], precision=hp) + b2_ref[...], 0.0)
        mu = jnp.dot(h, wmu_ref[...], precision=hp) + bmu_ref[...]
        lv = jnp.dot(h, wlv_ref[...], precision=hp) + blv_ref[...]
        z = mu + eps_ref[...] * jnp.exp(0.5 * lv)
        z_ref[...] = z
        mu_ref[...] = mu
        lv_ref[...] = lv

    rep = lambda shape: pl.BlockSpec(shape, lambda i: (0,) * len(shape))
    out_sds = jax.ShapeDtypeStruct((B, LAT), jnp.float32)
    return pl.pallas_call(
        body,
        grid=(GRID,),
        in_specs=[
            pl.BlockSpec((RPB, 128), lambda i: (i, 0)),
            pl.BlockSpec((1, 8, RPB), lambda i: (i, 0, 0)),
            rep((128, 8 * K_DIM)), rep((1, 8 * K_DIM)),
            rep((K_DIM, H1)), rep((1, H1)),
            rep((H1, H2)), rep((1, H2)),
            rep((H2, LAT)), rep((1, LAT)),
            rep((H2, LAT)), rep((1, LAT)),
            pl.BlockSpec((BB, LAT), lambda i: (i, 0)),
        ],
        out_specs=[
            pl.BlockSpec((BB, LAT), lambda i: (i, 0)),
            pl.BlockSpec((BB, LAT), lambda i: (i, 0)),
            pl.BlockSpec((BB, LAT), lambda i: (i, 0)),
        ],
        out_shape=[out_sds, out_sds, out_sds],
    )(e8, mjs3, wbig, bbig, w1, b1, w2, b2, wmu, bmu, wlv, blv, eps)


def kernel(x, mask, table, W_pnnn, b_pnnn, W1, b1, W2, b2, Wmu, bmu, Wlv, blv, eps):
    x32 = x.reshape(NW, NCH, CHUNK)
    table_lin = _sc_pack(table.T).reshape(VP, EMB)
    e8 = _sc_gather(x32, table_lin).reshape(TOT // 8, 128)
    mjs3 = mask.astype(jnp.float32).reshape(GRID, RPB, 8).transpose(0, 2, 1)
    wbig = jnp.kron(jnp.eye(8, dtype=jnp.float32), W_pnnn.T)
    bbig = jnp.tile(b_pnnn, 8).reshape(1, 8 * K_DIM)
    z, mu, lv = _tc_fused(
        e8, mjs3, wbig, bbig,
        W1.T, b1.reshape(1, H1),
        W2.T, b2.reshape(1, H2),
        Wmu.T, bmu.reshape(1, LAT),
        Wlv.T, blv.reshape(1, LAT),
        eps,
    )
    return (z, mu, lv)


# speedup vs baseline: 7.5091x; 1.9896x over previous
import functools

import jax
import jax.numpy as jnp
from jax import lax
from jax.experimental import pallas as pl
from jax.experimental.pallas import tpu as pltpu
from jax.experimental.pallas import tpu_sc as plsc

B, L = 1024, 100
EMB = 16
K_DIM, H1, H2, LAT = 64, 128, 64, 32

NC, NS = 2, 16
NW = NC * NS
TOT = B * L
PER_W = TOT // NW
CHUNK = 128
NCH = PER_W // CHUNK

BB = 64
GRID = B // BB

V = 1000000
CC = 8192


NP_PAIRS = 7813
VP = 128 * NP_PAIRS
NBUF = 8
TPW = (NP_PAIRS + NW - 1) // NW


def _sc_pack(table_t):
    mesh = plsc.VectorSubcoreMesh(core_axis_name="c", subcore_axis_name="s")

    @functools.partial(
        pl.kernel,
        out_type=jax.ShapeDtypeStruct((VP // 8, 128), jnp.float32),
        mesh=mesh,
        scratch_types=[
            pltpu.VMEM((NBUF, 16, 128), jnp.float32),
            pltpu.VMEM((NBUF, 16, 128), jnp.float32),
            pltpu.SemaphoreType.DMA((NBUF,)),
            pltpu.SemaphoreType.DMA((NBUF,)),
        ],
        compiler_params=pltpu.CompilerParams(
            use_tc_tiling_on_sc=True, needs_layout_passes=False),
    )
    def pack_kernel(tt_hbm, out_hbm, in_v, rows_v, sem_in, sem_out):
        wid = lax.axis_index("s") * NC + lax.axis_index("c")
        m = lax.iota(jnp.int32, 16)
        rv = [2 * k + m // 8 for k in range(8)]
        cv = [(m % 8) * 16 + d for d in range(16)]

        def in_desc(b, c):
            return pltpu.make_async_copy(
                tt_hbm.at[:, pl.ds(128 * c, 128)], in_v.at[b], sem_in.at[b])

        def out_desc(b, c):
            return pltpu.make_async_copy(
                rows_v.at[b], out_hbm.at[pl.ds(16 * c, 16)], sem_out.at[b])

        for b in range(NBUF):
            in_desc(b, wid + NW * b).start()

        def step(t, carry):
            b = lax.rem(t, NBUF)
            c = wid + NW * t

            @pl.when(c < NP_PAIRS)
            def _():
                in_desc(b, c).wait()

                @pl.when(t >= NBUF)
                def _():
                    out_desc(b, c - NW * NBUF).wait()

                for d in range(16):
                    for k in range(8):
                        vals = in_v[b, d, pl.ds(16 * k, 16)]
                        plsc.store_scatter(rows_v.at[b], [rv[k], cv[d]], vals)
                out_desc(b, c).start()

                @pl.when(c + NW * NBUF < NP_PAIRS)
                def _():
                    in_desc(b, c + NW * NBUF).start()
            return carry

        lax.fori_loop(0, TPW, step, 0, unroll=False)

        t_max = (NP_PAIRS - 1 - wid) // NW
        for b in range(NBUF):
            t_last = t_max - lax.rem(t_max - b + NBUF, NBUF)
            out_desc(b, wid + NW * t_last).wait()

    return pack_kernel(table_t)


def _sc_gather(x32, table):
    mesh = plsc.VectorSubcoreMesh(core_axis_name="c", subcore_axis_name="s")

    @functools.partial(
        pl.kernel,
        out_type=jax.ShapeDtypeStruct((NW, PER_W, EMB), jnp.float32),
        mesh=mesh,
        scratch_types=[
            pltpu.VMEM((NCH, CHUNK), jnp.int32),
            pltpu.VMEM((PER_W, EMB), jnp.float32),
            pltpu.SemaphoreType.DMA,
        ],
        compiler_params=pltpu.CompilerParams(use_tc_tiling_on_sc=False),
    )
    def gather_kernel(x_hbm, table_hbm, out_hbm, idx_v, rows_v, sem):
        wid = lax.axis_index("s") * NC + lax.axis_index("c")
        pltpu.sync_copy(x_hbm.at[wid], idx_v)

        def fire(j, _):
            pltpu.make_async_copy(
                table_hbm.at[idx_v.at[j]],
                rows_v.at[pl.ds(j * CHUNK, CHUNK)],
                sem,
            ).start()
            return _

        lax.fori_loop(0, NCH, fire, 0, unroll=False)

        def drain(j, _):
            pltpu.make_async_copy(
                table_hbm.at[idx_v.at[j]],
                rows_v.at[pl.ds(j * CHUNK, CHUNK)],
                sem,
            ).wait()
            return _

        lax.fori_loop(0, NCH, drain, 0, unroll=False)
        pltpu.sync_copy(rows_v, out_hbm.at[wid])

    return gather_kernel(x32, table)


RPB = BB * L // 8


def _tc_fused(e8, mjs3, wbig, bbig, w1, b1, w2, b2, wmu, bmu, wlv, blv, eps):
    def body(e_ref, m_ref, wb_ref, bb_ref, w1_ref, b1_ref, w2_ref, b2_ref,
             wmu_ref, bmu_ref, wlv_ref, blv_ref, eps_ref,
             z_ref, mu_ref, lv_ref):
        hp = jax.lax.Precision.HIGHEST
        p8 = jnp.dot(e_ref[...], wb_ref[...]) + bb_ref[...]
        p8 = jnp.maximum(p8, 0.0)
        bb_lo = jax.lax.broadcasted_iota(jnp.int32, (BB, RPB), 0) * L
        el8 = jax.lax.broadcasted_iota(jnp.int32, (BB, RPB), 1) * 8
        pnc = jnp.zeros((BB, K_DIM), jnp.float32)
        for j in range(8):
            el = el8 + j
            sel = ((el >= bb_lo) & (el < bb_lo + L)).astype(jnp.float32)
            sel = sel * m_ref[0, j, :][None, :]
            pnc = pnc + jnp.dot(sel, p8[:, 64 * j:64 * (j + 1)])
        h = jnp.maximum(jnp.dot(pnc, w1_ref[...], precision=hp) + b1_ref[...], 0.0)
        h = jnp.maximum(jnp.dot(h, w2_ref[...], precision=hp) + b2_ref[...], 0.0)
        mu = jnp.dot(h, wmu_ref[...], precision=hp) + bmu_ref[...]
        lv = jnp.dot(h, wlv_ref[...], precision=hp) + blv_ref[...]
        z = mu + eps_ref[...] * jnp.exp(0.5 * lv)
        z_ref[...] = z
        mu_ref[...] = mu
        lv_ref[...] = lv

    rep = lambda shape: pl.BlockSpec(shape, lambda i: (0,) * len(shape))
    out_sds = jax.ShapeDtypeStruct((B, LAT), jnp.float32)
    return pl.pallas_call(
        body,
        grid=(GRID,),
        in_specs=[
            pl.BlockSpec((RPB, 128), lambda i: (i, 0)),
            pl.BlockSpec((1, 8, RPB), lambda i: (i, 0, 0)),
            rep((128, 8 * K_DIM)), rep((1, 8 * K_DIM)),
            rep((K_DIM, H1)), rep((1, H1)),
            rep((H1, H2)), rep((1, H2)),
            rep((H2, LAT)), rep((1, LAT)),
            rep((H2, LAT)), rep((1, LAT)),
            pl.BlockSpec((BB, LAT), lambda i: (i, 0)),
        ],
        out_specs=[
            pl.BlockSpec((BB, LAT), lambda i: (i, 0)),
            pl.BlockSpec((BB, LAT), lambda i: (i, 0)),
            pl.BlockSpec((BB, LAT), lambda i: (i, 0)),
        ],
        out_shape=[out_sds, out_sds, out_sds],
    )(e8, mjs3, wbig, bbig, w1, b1, w2, b2, wmu, bmu, wlv, blv, eps)


def kernel(x, mask, table, W_pnnn, b_pnnn, W1, b1, W2, b2, Wmu, bmu, Wlv, blv, eps):
    x32 = x.reshape(NW, NCH, CHUNK)
    table_lin = _sc_pack(table.T).reshape(VP, EMB)
    e8 = _sc_gather(x32, table_lin).reshape(TOT // 8, 128)
    mjs3 = mask.astype(jnp.float32).reshape(GRID, RPB, 8).transpose(0, 2, 1)
    wbig = jnp.kron(jnp.eye(8, dtype=jnp.float32), W_pnnn.T)
    bbig = jnp.tile(b_pnnn, 8).reshape(1, 8 * K_DIM)
    z, mu, lv = _tc_fused(
        e8, mjs3, wbig, bbig,
        W1.T, b1.reshape(1, H1),
        W2.T, b2.reshape(1, H2),
        Wmu.T, bmu.reshape(1, LAT),
        Wlv.T, blv.reshape(1, LAT),
        eps,
    )
    return (z, mu, lv)
